# Initial kernel scaffold; baseline (speedup 1.0000x reference)
import jax
import jax.numpy as jnp
from jax.experimental import pallas as pl


def kernel(states, emb_c, emb_s, W1, b1, W2, b2, W3, b3):
    raise NotImplementedError("write your pallas kernel here")


# speedup vs baseline: 50.7498x; 50.7498x over previous
import functools

import jax
import jax.numpy as jnp
from jax import lax
from jax.experimental import pallas as pl
from jax.experimental.pallas import tpu as pltpu
from jax.experimental.pallas import tpu_sc as plsc

SC_CORES = 2
SC_SUBCORES = 16
SC_LANES = 16


def _score_body(emb_c_ref, emb_s_ref, w1_ref, b1_ref, w2_ref, b2_ref,
                w3_ref, out_ref):
    cb, h = emb_c_ref.shape
    spw = emb_s_ref.shape[0]
    e = emb_c_ref[...][:, None, :] + emb_s_ref[...][None, :, :]
    e = e.reshape(cb * spw, h)
    hid = jnp.maximum(
        jnp.dot(e, w1_ref[...], preferred_element_type=jnp.float32)
        + b1_ref[...], 0.0)
    r = jnp.maximum(
        jnp.dot(hid, w2_ref[...], preferred_element_type=jnp.float32)
        + b2_ref[...], 0.0) + e
    out_ref[...] = jnp.sum(r * w3_ref[...], axis=1, keepdims=True)


def _score_table(emb_c, emb_s, W1, b1, W2, b2, W3):
    num_clusters, h = emb_c.shape
    spw = emb_s.shape[0]
    grid = 4
    cb = num_clusters // grid
    out = pl.pallas_call(
        _score_body,
        grid=(grid,),
        in_specs=[
            pl.BlockSpec((cb, h), lambda i: (i, 0)),
            pl.BlockSpec((spw, h), lambda i: (0, 0)),
            pl.BlockSpec((h, h), lambda i: (0, 0)),
            pl.BlockSpec((1, h), lambda i: (0, 0)),
            pl.BlockSpec((h, h), lambda i: (0, 0)),
            pl.BlockSpec((1, h), lambda i: (0, 0)),
            pl.BlockSpec((1, h), lambda i: (0, 0)),
        ],
        out_specs=pl.BlockSpec((cb * spw, 1), lambda i: (i, 0)),
        out_shape=jax.ShapeDtypeStruct((num_clusters * spw, 1), jnp.float32),
    )(emb_c, emb_s, W1, b1.reshape(1, h), W2, b2.reshape(1, h),
      W3.reshape(1, h))
    return out.reshape(num_clusters * spw)


def _sc_gather(score, idx_flat):
    n = idx_flat.shape[0]
    c = score.shape[0]
    nw = SC_CORES * SC_SUBCORES
    epw = n // nw

    mesh = plsc.VectorSubcoreMesh(core_axis_name="c", subcore_axis_name="s")

    @functools.partial(
        pl.kernel,
        mesh=mesh,
        out_type=jax.ShapeDtypeStruct((n,), jnp.float32),
        scratch_types=[
            pltpu.VMEM((c,), jnp.float32),
            pltpu.VMEM((epw,), jnp.int32),
            pltpu.VMEM((epw,), jnp.float32),
        ],
        compiler_params=pltpu.CompilerParams(needs_layout_passes=False),
    )
    def gather_kernel(score_hbm, idx_hbm, out_hbm, table_v, idx_v, vals_v):
        wid = lax.axis_index("s") * SC_CORES + lax.axis_index("c")
        base = wid * epw
        pltpu.sync_copy(score_hbm, table_v)
        pltpu.sync_copy(idx_hbm.at[pl.ds(base, epw)], idx_v)

        def body(i, carry):
            off = i * SC_LANES
            idx = idx_v[pl.ds(off, SC_LANES)]
            vals_v[pl.ds(off, SC_LANES)] = plsc.load_gather(table_v, [idx])
            return carry

        lax.fori_loop(0, epw // SC_LANES, body, 0, unroll=8)
        pltpu.sync_copy(vals_v, out_hbm.at[pl.ds(base, epw)])

    return gather_kernel(score, idx_flat)


def _lsm_body(x_ref, o_ref):
    x = x_ref[...]
    m = jnp.max(x, axis=-1, keepdims=True)
    ex = jnp.exp(x - m)
    o_ref[...] = x - m - jnp.log(jnp.sum(ex, axis=-1, keepdims=True))


def _log_softmax(logits):
    return pl.pallas_call(
        _lsm_body,
        out_shape=jax.ShapeDtypeStruct(logits.shape, jnp.float32),
    )(logits)


def kernel(states, emb_c, emb_s, W1, b1, W2, b2, W3, b3):
    b, spw = states.shape
    score = _score_table(emb_c, emb_s, W1, b1, W2, b2, W3)
    idx_flat = states.reshape(b * spw).astype(jnp.int32)
    logits = _sc_gather(score, idx_flat).reshape(b, spw)
    return _log_softmax(logits)
